# Initial kernel scaffold; baseline (speedup 1.0000x reference)
import jax
import jax.numpy as jnp
from jax.experimental import pallas as pl


def kernel(features, edges, edges2, edge_features, additional_feature, W1, b1, W2, b2):
    raise NotImplementedError("write your pallas kernel here")


# speedup vs baseline: 13.4869x; 13.4869x over previous
import functools

import jax
import jax.numpy as jnp
from jax import lax
from jax.experimental import pallas as pl
from jax.experimental.pallas import tpu as pltpu
from jax.experimental.pallas import tpu_sc as plsc

NC = 2
NS = 16
L = 16
NW = NC * NS
K = 128
DEGW = 16


def _mesh():
    return plsc.VectorSubcoreMesh(
        core_axis_name="c", subcore_axis_name="s",
        num_cores=NC, num_subcores=NS)


@functools.lru_cache(maxsize=None)
def _make_kdeg(NPAD, C):
    @functools.partial(
        pl.kernel,
        out_type=jax.ShapeDtypeStruct((NC, NPAD, DEGW), jnp.float32),
        mesh=_mesh(),
        scratch_types=[
            pltpu.VMEM((C, K), jnp.int32),
            pltpu.VMEM((K, DEGW), jnp.float32),
            pltpu.VMEM_SHARED((NPAD, DEGW), jnp.float32),
        ],
    )
    def kdeg(dst_hbm, zeros_hbm, out_hbm, dst_v, ones_v, acc):
        c = lax.axis_index("c")
        s = lax.axis_index("s")
        w = s * NC + c

        def fill(r, carry):
            ones_v[r, pl.ds(0, L)] = jnp.ones((L,), jnp.float32)
            return carry
        lax.fori_loop(0, K, fill, 0)

        @pl.when(s == 0)
        def _():
            pltpu.sync_copy(zeros_hbm, acc)
        pltpu.sync_copy(dst_hbm.at[w], dst_v)
        plsc.subcore_barrier()

        def body(j, carry):
            pltpu.sync_copy(ones_v, acc.at[dst_v.at[j]], add=True)
            return carry
        lax.fori_loop(0, C, body, 0)

        plsc.subcore_barrier()

        @pl.when(s == 0)
        def _():
            pltpu.sync_copy(acc, out_hbm.at[c])

    return kdeg


@functools.lru_cache(maxsize=None)
def _make_kagg(NPAD, D, C):
    @functools.partial(
        pl.kernel,
        out_type=jax.ShapeDtypeStruct((NC, NPAD, D), jnp.float32),
        mesh=_mesh(),
        scratch_types=[
            pltpu.VMEM((C, K), jnp.int32),
            pltpu.VMEM((C, K), jnp.int32),
            pltpu.VMEM((K, D), jnp.float32),
            pltpu.VMEM_SHARED((NPAD, D), jnp.float32),
            pltpu.SemaphoreType.DMA,
        ],
    )
    def kagg(y_hbm, src_hbm, dst_hbm, zeros_hbm, out_hbm,
             src_v, dst_v, rows_v, acc, sem):
        c = lax.axis_index("c")
        s = lax.axis_index("s")
        w = s * NC + c

        @pl.when(s == 0)
        def _():
            pltpu.sync_copy(zeros_hbm, acc)
        pltpu.sync_copy(src_hbm.at[w], src_v)
        pltpu.sync_copy(dst_hbm.at[w], dst_v)
        plsc.subcore_barrier()

        def body(j, carry):
            pltpu.async_copy(y_hbm.at[src_v.at[j]], rows_v, sem).wait()
            pltpu.sync_copy(rows_v, acc.at[dst_v.at[j]], add=True)
            return carry
        lax.fori_loop(0, C, body, 0)

        plsc.subcore_barrier()

        @pl.when(s == 0)
        def _():
            pltpu.sync_copy(acc, out_hbm.at[c])

    return kagg


def _mm(x, W, B=2048):
    NPAD, D = x.shape

    def body(x_ref, w_ref, o_ref):
        o_ref[...] = jnp.dot(x_ref[...], w_ref[...],
                             preferred_element_type=jnp.float32)

    return pl.pallas_call(
        body,
        grid=(NPAD // B,),
        in_specs=[pl.BlockSpec((B, D), lambda i: (i, 0)),
                  pl.BlockSpec((D, D), lambda i: (0, 0))],
        out_specs=pl.BlockSpec((B, D), lambda i: (i, 0)),
        out_shape=jax.ShapeDtypeStruct((NPAD, D), jnp.float32),
    )(x, W)


def _prep(d0, d1, xw, B=2048):
    NPAD, D = xw.shape

    def body(d0_ref, d1_ref, xw_ref, y_ref, dis_ref):
        deg = d0_ref[:, 0:1] + d1_ref[:, 0:1] + 1.0
        dis = lax.rsqrt(deg)
        y_ref[...] = dis * xw_ref[...]
        dis_ref[...] = jnp.broadcast_to(dis, xw_ref.shape)

    return pl.pallas_call(
        body,
        grid=(NPAD // B,),
        in_specs=[pl.BlockSpec((B, DEGW), lambda i: (i, 0)),
                  pl.BlockSpec((B, DEGW), lambda i: (i, 0)),
                  pl.BlockSpec((B, D), lambda i: (i, 0))],
        out_specs=[pl.BlockSpec((B, D), lambda i: (i, 0)),
                   pl.BlockSpec((B, D), lambda i: (i, 0))],
        out_shape=[jax.ShapeDtypeStruct((NPAD, D), jnp.float32),
                   jax.ShapeDtypeStruct((NPAD, D), jnp.float32)],
    )(d0, d1, xw)


def _mid(a0, a1, y1, dis, W2, b1, B=2048):
    NPAD, D = y1.shape

    def body(a0_ref, a1_ref, y1_ref, dis_ref, w_ref, b_ref, y2_ref):
        h = dis_ref[...] * (a0_ref[...] + a1_ref[...] + y1_ref[...]) + b_ref[...]
        h = jnp.maximum(h, 0.0)
        y2_ref[...] = dis_ref[...] * jnp.dot(h, w_ref[...],
                                             preferred_element_type=jnp.float32)

    return pl.pallas_call(
        body,
        grid=(NPAD // B,),
        in_specs=[pl.BlockSpec((B, D), lambda i: (i, 0)),
                  pl.BlockSpec((B, D), lambda i: (i, 0)),
                  pl.BlockSpec((B, D), lambda i: (i, 0)),
                  pl.BlockSpec((B, D), lambda i: (i, 0)),
                  pl.BlockSpec((D, D), lambda i: (0, 0)),
                  pl.BlockSpec((1, D), lambda i: (0, 0))],
        out_specs=pl.BlockSpec((B, D), lambda i: (i, 0)),
        out_shape=jax.ShapeDtypeStruct((NPAD, D), jnp.float32),
    )(a0, a1, y1, dis, W2, b1)


def _fin(a0, a1, y2, dis, b2, N, B=2000):
    NPAD, D = y2.shape

    def body(a0_ref, a1_ref, y2_ref, dis_ref, b_ref, o_ref):
        o_ref[...] = (dis_ref[...] * (a0_ref[...] + a1_ref[...] + y2_ref[...])
                      + b_ref[...])

    return pl.pallas_call(
        body,
        grid=(N // B,),
        in_specs=[pl.BlockSpec((B, D), lambda i: (i, 0)),
                  pl.BlockSpec((B, D), lambda i: (i, 0)),
                  pl.BlockSpec((B, D), lambda i: (i, 0)),
                  pl.BlockSpec((B, D), lambda i: (i, 0)),
                  pl.BlockSpec((1, D), lambda i: (0, 0))],
        out_specs=pl.BlockSpec((B, D), lambda i: (i, 0)),
        out_shape=jax.ShapeDtypeStruct((N, D), jnp.float32),
    )(a0, a1, y2, dis, b2)


def kernel(features, edges, edges2, edge_features, additional_feature,
           W1, b1, W2, b2):
    N, D = features.shape
    E = edges2.shape[1]

    NPAD = -(-(N + 1) // (NS * K)) * (NS * K)
    C = -(-E // (NW * K))
    EPAD = C * K * NW

    src = edges2[0]
    dst = edges2[1]
    padv = jnp.full((EPAD - E,), N, jnp.int32)
    src3 = jnp.concatenate([src, padv]).reshape(NW, C, K)
    dst3 = jnp.concatenate([dst, padv]).reshape(NW, C, K)
    x_pad = jnp.pad(features, ((0, NPAD - N), (0, 0)))
    b1r = b1.reshape(1, D)
    b2r = b2.reshape(1, D)

    kdeg = _make_kdeg(NPAD, C)
    kagg = _make_kagg(NPAD, D, C)
    zdeg = jnp.zeros((NPAD, DEGW), jnp.float32)
    zagg = jnp.zeros((NPAD, D), jnp.float32)

    deg_p = kdeg(dst3, zdeg)
    xw1 = _mm(x_pad, W1)
    y1, dis = _prep(deg_p[0], deg_p[1], xw1)
    ag1 = kagg(y1, src3, dst3, zagg)
    y2 = _mid(ag1[0], ag1[1], y1, dis, W2, b1r)
    ag2 = kagg(y2, src3, dst3, zagg)
    return _fin(ag2[0], ag2[1], y2, dis, b2r, N)
